# baseline (speedup 1.0000x reference)
import jax
import jax.numpy as jnp
from jax.experimental import pallas as pl
from jax.experimental.pallas import tpu as pltpu

C = 128
H = 256
F = 256
T = 4
IT = 2
NSEM = 57
NSEM_PAD = 64

BI = 32
NBLK = C // BI
PF_BN = 4096
PF_ROWS = PF_BN // H
N_A = (H * C) // PF_BN
BIG = 1e30

_S_A = 0
_S_B1 = _S_A + N_A
_S_B2 = _S_B1 + 1
_S_MP0 = _S_B2 + NBLK
_S_MP1 = _S_MP0 + NBLK
_S_D = _S_MP1 + NBLK
_NSTEPS = _S_D + 1


def _dot(a, b):
    return jnp.dot(a, b, preferred_element_type=jnp.float32)


def _mp_stage(i, cf_s, el_s, coeff_s, cnt_s,
              w1_ref, w2_ref, w3_ref, w4big_ref, bne_ref, out_s):
    i0 = i * BI
    cf = cf_s[...]
    a_rows = cf_s[pl.ds(i0, BI), :]
    a_msg = _dot(a_rows, w1_ref[...])
    b_msg = _dot(cf, w2_ref[...]) + bne_ref[...]
    el = el_s[pl.ds(i0 * C, BI * C), :]
    elw = _dot(el, w3_ref[...]).reshape(BI, C, H)
    base = (elw + a_msg[:, None, :] + b_msg[None, :, :]).reshape(BI * C, H)
    coeff = coeff_s[pl.ds(i0 * C, BI * C), :]
    mfm1 = jnp.where(coeff > 0, 0.0, -1.0)
    nt = jnp.zeros((BI * C, H), dtype=jnp.float32)
    for t in range(T):
        lhs = jnp.concatenate([coeff[:, t:t + 1], mfm1[:, t:t + 1]],
                              axis=1).astype(jnp.bfloat16)
        z_t = _dot(lhs, w4big_ref[pl.ds(2 * t, 2), :])
        nt = nt + jnp.maximum(base + z_t, 0.0)
    rows = jax.lax.broadcasted_iota(jnp.int32, (BI, BI * C), 0)
    cols = jax.lax.broadcasted_iota(jnp.int32, (BI, BI * C), 1)
    sel = (cols // C == rows).astype(jnp.float32)
    sums = _dot(sel, nt)
    denom = jnp.maximum(cnt_s[pl.ds(i0, BI), :], 1.0)
    cf_new = sums / denom
    total = jnp.sum(cnt_s[...])
    out_s[pl.ds(i0, BI), :] = jnp.where(total > 0.0, cf_new, a_rows)


def _body(parent_ref, wp_ref, bp_ref, we_ref, be_ref, wela_ref, welb_ref,
          bel_ref, wee_ref, bee_ref,
          w1a_ref, w2a_ref, w3a_ref, w4biga_ref, bnea_ref,
          w1b_ref, w2b_ref, w3b_ref, w4bigb_ref, bneb_ref,
          wc_ref, bc_ref, wsem_ref, bsem_ref, wc2_ref, bc2_ref,
          cel_ref, eel_ref, feat_ref, sem_ref,
          cf0_s, u_s, v_s, cel_s, el_s, coeff_s, cnt_s, cf1_s, cf2_s):
    s = pl.program_id(0)

    @pl.when(s < _S_B1)
    def _stage_a():
        o = jnp.maximum(_dot(parent_ref[...], wp_ref[...]) + bp_ref[...], 0.0)
        cf0_s[pl.ds(s * PF_ROWS, PF_ROWS), :] = o.reshape(PF_ROWS, H)

    @pl.when(s == _S_B1)
    def _stage_b1():
        cf = cf0_s[...]
        cel = _dot(cf, we_ref[...]) + be_ref[...]
        cel_s[...] = cel
        cel_ref[...] = cel
        u_s[...] = _dot(cf, wela_ref[...])
        v_s[...] = _dot(cf, welb_ref[...]) + bel_ref[...]

    @pl.when((s >= _S_B2) & (s < _S_MP0))
    def _stage_b2():
        i = s - _S_B2
        i0 = i * BI
        u_b = u_s[pl.ds(i0, BI), :]
        el = jnp.maximum(u_b[:, None, :] + v_s[...][None, :, :],
                         0.0).reshape(BI * C, H)
        el_s[pl.ds(i0 * C, BI * C), :] = el.astype(jnp.bfloat16)
        eel = _dot(el, wee_ref[...]) + bee_ref[...]
        eel_ref[...] = eel
        cel = cel_s[...]
        celj = jnp.broadcast_to(cel.reshape(1, C, 1),
                                (BI, C, 1)).reshape(BI * C, 1)
        celi = jnp.broadcast_to(cel_s[pl.ds(i0, BI), :][:, None, :],
                                (BI, C, 1)).reshape(BI * C, 1)
        mask = (eel > 0) & (celi > 0) & (celj > 0)
        mf = mask.astype(jnp.float32)
        coeff_s[pl.ds(i0 * C, BI * C), :] = mf * eel
        s1 = jnp.sum(mf.reshape(BI, C, T), axis=2)
        cnt_s[pl.ds(i0, BI), :] = jnp.sum(s1, axis=1)[:, None]

    @pl.when((s >= _S_MP0) & (s < _S_MP1))
    def _stage_mp0():
        _mp_stage(s - _S_MP0, cf0_s, el_s, coeff_s, cnt_s,
                  w1a_ref, w2a_ref, w3a_ref, w4biga_ref, bnea_ref, cf1_s)

    @pl.when((s >= _S_MP1) & (s < _S_D))
    def _stage_mp1():
        _mp_stage(s - _S_MP1, cf1_s, el_s, coeff_s, cnt_s,
                  w1b_ref, w2b_ref, w3b_ref, w4bigb_ref, bneb_ref, cf2_s)

    @pl.when(s == _S_D)
    def _stage_d():
        y = (_dot(cf0_s[...], wc_ref[0:H, :])
             + _dot(cf1_s[...], wc_ref[H:2 * H, :])
             + _dot(cf2_s[...], wc_ref[2 * H:3 * H, :])
             + bc_ref[...])
        y = jnp.maximum(y, 0.0)
        sem_ref[...] = _dot(y, wsem_ref[...]) + bsem_ref[...]
        feat_ref[...] = jnp.maximum(_dot(y, wc2_ref[...]) + bc2_ref[...], 0.0)


def kernel(parent_feature, Wp, bp, We, be, Wel, bel, Wee, bee, Wne, bne,
           Wc, bc, Wsem, bsem, Wc2, bc2):
    wsem_p = jnp.pad(Wsem, ((0, 0), (0, NSEM_PAD - NSEM)))
    bsem_p = jnp.pad(bsem, (0, NSEM_PAD - NSEM)).reshape(1, NSEM_PAD)
    big_row = jnp.full((1, H), BIG, dtype=jnp.float32)

    def wne_slices(i):
        w4big = jnp.concatenate(
            [jnp.stack([Wne[i, 3 * H + t], big_row[0]]) for t in range(T)], 0)
        return (Wne[i, 0:H], Wne[i, H:2 * H],
                Wne[i, 2 * H:3 * H].astype(jnp.bfloat16),
                w4big.astype(jnp.bfloat16), bne[i].reshape(1, H))

    w1a, w2a, w3a, w4biga, bnea = wne_slices(0)
    w1b, w2b, w3b, w4bigb, bneb = wne_slices(1)

    full = lambda shp: pl.BlockSpec(shp, lambda s: tuple(0 for _ in shp))
    wp_spec = pl.BlockSpec((F, PF_BN), lambda s: (0, jnp.minimum(s, N_A - 1)))
    bp_spec = pl.BlockSpec((1, PF_BN), lambda s: (0, jnp.minimum(s, N_A - 1)))
    eel_spec = pl.BlockSpec(
        (BI * C, T), lambda s: (jnp.clip(s - _S_B2, 0, NBLK - 1), 0))

    cel, eel, feats, sem = pl.pallas_call(
        _body,
        grid=(_NSTEPS,),
        in_specs=[
            full((1, F)), wp_spec, bp_spec,
            full((H, 1)), full((1, 1)), full((H, H)), full((H, H)),
            full((1, H)), full((H, T)), full((1, T)),
            full((H, H)), full((H, H)), full((H, H)), full((2 * T, H)),
            full((1, H)),
            full((H, H)), full((H, H)), full((H, H)), full((2 * T, H)),
            full((1, H)),
            full((3 * H, H)), full((1, H)), full((H, NSEM_PAD)),
            full((1, NSEM_PAD)), full((H, F)), full((1, F)),
        ],
        out_specs=[
            full((C, 1)), eel_spec, full((C, F)), full((C, NSEM_PAD)),
        ],
        out_shape=[
            jax.ShapeDtypeStruct((C, 1), jnp.float32),
            jax.ShapeDtypeStruct((C * C, T), jnp.float32),
            jax.ShapeDtypeStruct((C, F), jnp.float32),
            jax.ShapeDtypeStruct((C, NSEM_PAD), jnp.float32),
        ],
        scratch_shapes=[
            pltpu.VMEM((C, H), jnp.float32),
            pltpu.VMEM((C, H), jnp.float32),
            pltpu.VMEM((C, H), jnp.float32),
            pltpu.VMEM((C, 1), jnp.float32),
            pltpu.VMEM((C * C, H), jnp.bfloat16),
            pltpu.VMEM((C * C, T), jnp.float32),
            pltpu.VMEM((C, 1), jnp.float32),
            pltpu.VMEM((C, H), jnp.float32),
            pltpu.VMEM((C, H), jnp.float32),
        ],
    )(parent_feature, Wp, bp.reshape(1, H * C), We, be.reshape(1, 1),
      Wel[0:H], Wel[H:2 * H], bel.reshape(1, H), Wee, bee.reshape(1, T),
      w1a, w2a, w3a, w4biga, bnea, w1b, w2b, w3b, w4bigb, bneb,
      Wc, bc.reshape(1, H), wsem_p, bsem_p, Wc2, bc2.reshape(1, F))

    return (feats.reshape(1, C, F),
            sem[:, :NSEM].reshape(1, C, NSEM),
            cel.reshape(1, C, 1),
            eel.reshape(1, C, C, T))


# speedup vs baseline: 1.0666x; 1.0666x over previous
import jax
import jax.numpy as jnp
from jax.experimental import pallas as pl
from jax.experimental.pallas import tpu as pltpu

C = 128
H = 256
F = 256
T = 4
IT = 2
NSEM = 57
NSEM_PAD = 64

BI = 32
NBLK = C // BI
PF_BN = 4096
PF_ROWS = PF_BN // H
N_A = (H * C) // PF_BN
BIG = 1e30

_S_A = 0
_S_B1 = _S_A + N_A
_S_B2 = _S_B1 + 1
_S_MP0 = _S_B2 + NBLK
_S_MP1 = _S_MP0 + NBLK
_S_D = _S_MP1 + NBLK
_NSTEPS = _S_D + 1


def _dot(a, b):
    return jnp.dot(a, b, preferred_element_type=jnp.float32)


def _mp_stage(i, it, cf_s, el_s, coeff_s, cnt_s, wne_ref, bne2_ref, out_s):
    i0 = i * BI
    cf = cf_s[...]
    a_rows = cf_s[pl.ds(i0, BI), :]
    a_msg = _dot(a_rows, wne_ref[it, 0:H, :])
    b_msg = _dot(cf, wne_ref[it, H:2 * H, :]) + bne2_ref[it:it + 1, :]
    el = el_s[pl.ds(i0 * C, BI * C), :]
    elw = _dot(el, wne_ref[it, 2 * H:3 * H, :]).reshape(BI, C, H)
    base = (elw + a_msg[:, None, :] + b_msg[None, :, :]).reshape(BI * C, H)
    coeff = coeff_s[pl.ds(i0 * C, BI * C), :]
    mfm1 = jnp.where(coeff > 0, 0.0, -1.0)
    nt = jnp.zeros((BI * C, H), dtype=jnp.float32)
    w4 = wne_ref[it, 3 * H:3 * H + T, :]
    bigrow = jnp.full((1, H), BIG, dtype=jnp.float32)
    for t in range(T):
        lhs = jnp.concatenate([coeff[:, t:t + 1], mfm1[:, t:t + 1]], axis=1)
        rhs = jnp.concatenate([w4[t:t + 1, :], bigrow], axis=0)
        z_t = _dot(lhs, rhs)
        nt = nt + jnp.maximum(base + z_t, 0.0)
    rows = jax.lax.broadcasted_iota(jnp.int32, (BI, BI * C), 0)
    cols = jax.lax.broadcasted_iota(jnp.int32, (BI, BI * C), 1)
    sel = (cols // C == rows).astype(jnp.float32)
    sums = _dot(sel, nt)
    denom = jnp.maximum(cnt_s[pl.ds(i0, BI), :], 1.0)
    cf_new = sums / denom
    total = jnp.sum(cnt_s[...])
    out_s[pl.ds(i0, BI), :] = jnp.where(total > 0.0, cf_new, a_rows)


def _body(parent_ref, wp_ref, bp_ref, we_ref, be_ref, wela_ref, welb_ref,
          bel_ref, wee_ref, bee_ref,
          wne_ref, bne2_ref,
          wc_ref, bc_ref, wsem_ref, bsem_ref, wc2_ref, bc2_ref,
          cel_ref, eel_ref, feat_ref, sem_ref,
          cf0_s, u_s, v_s, cel_s, el_s, coeff_s, cnt_s, cf1_s, cf2_s):
    s = pl.program_id(0)

    @pl.when(s < _S_B1)
    def _stage_a():
        o = jnp.maximum(_dot(parent_ref[...], wp_ref[...]) + bp_ref[...], 0.0)
        cf0_s[pl.ds(s * PF_ROWS, PF_ROWS), :] = o.reshape(PF_ROWS, H)

    @pl.when(s == _S_B1)
    def _stage_b1():
        cf = cf0_s[...]
        cel = _dot(cf, we_ref[...]) + be_ref[...]
        cel_s[...] = cel
        cel_ref[...] = cel
        u_s[...] = _dot(cf, wela_ref[...])
        v_s[...] = _dot(cf, welb_ref[...]) + bel_ref[...]

    @pl.when((s >= _S_B2) & (s < _S_MP0))
    def _stage_b2():
        i = s - _S_B2
        i0 = i * BI
        u_b = u_s[pl.ds(i0, BI), :]
        el = jnp.maximum(u_b[:, None, :] + v_s[...][None, :, :],
                         0.0).reshape(BI * C, H)
        el_s[pl.ds(i0 * C, BI * C), :] = el
        eel = _dot(el, wee_ref[...]) + bee_ref[...]
        eel_ref[...] = eel
        cel = cel_s[...]
        celj = jnp.broadcast_to(cel.reshape(1, C, 1),
                                (BI, C, 1)).reshape(BI * C, 1)
        celi = jnp.broadcast_to(cel_s[pl.ds(i0, BI), :][:, None, :],
                                (BI, C, 1)).reshape(BI * C, 1)
        mask = (eel > 0) & (celi > 0) & (celj > 0)
        mf = mask.astype(jnp.float32)
        coeff_s[pl.ds(i0 * C, BI * C), :] = mf * eel
        s1 = jnp.sum(mf.reshape(BI, C, T), axis=2)
        cnt_s[pl.ds(i0, BI), :] = jnp.sum(s1, axis=1)[:, None]

    @pl.when((s >= _S_MP0) & (s < _S_MP1))
    def _stage_mp0():
        _mp_stage(s - _S_MP0, 0, cf0_s, el_s, coeff_s, cnt_s,
                  wne_ref, bne2_ref, cf1_s)

    @pl.when((s >= _S_MP1) & (s < _S_D))
    def _stage_mp1():
        _mp_stage(s - _S_MP1, 1, cf1_s, el_s, coeff_s, cnt_s,
                  wne_ref, bne2_ref, cf2_s)

    @pl.when(s == _S_D)
    def _stage_d():
        y = (_dot(cf0_s[...], wc_ref[0:H, :])
             + _dot(cf1_s[...], wc_ref[H:2 * H, :])
             + _dot(cf2_s[...], wc_ref[2 * H:3 * H, :])
             + bc_ref[...])
        y = jnp.maximum(y, 0.0)
        sem_ref[...] = _dot(y, wsem_ref[...]) + bsem_ref[...]
        feat_ref[...] = jnp.maximum(_dot(y, wc2_ref[...]) + bc2_ref[...], 0.0)


def kernel(parent_feature, Wp, bp, We, be, Wel, bel, Wee, bee, Wne, bne,
           Wc, bc, Wsem, bsem, Wc2, bc2):
    wsem_p = jnp.pad(Wsem, ((0, 0), (0, NSEM_PAD - NSEM)))
    bsem_p = jnp.pad(bsem, (0, NSEM_PAD - NSEM)).reshape(1, NSEM_PAD)

    full = lambda shp: pl.BlockSpec(shp, lambda s: tuple(0 for _ in shp))
    wp_spec = pl.BlockSpec((F, PF_BN), lambda s: (0, jnp.minimum(s, N_A - 1)))
    bp_spec = pl.BlockSpec((1, PF_BN), lambda s: (0, jnp.minimum(s, N_A - 1)))
    eel_spec = pl.BlockSpec(
        (BI * C, T), lambda s: (jnp.clip(s - _S_B2, 0, NBLK - 1), 0))

    cel, eel, feats, sem = pl.pallas_call(
        _body,
        grid=(_NSTEPS,),
        in_specs=[
            full((1, F)), wp_spec, bp_spec,
            full((H, 1)), full((1, 1)), full((H, H)), full((H, H)),
            full((1, H)), full((H, T)), full((1, T)),
            full((IT, 3 * H + T, H)), full((IT, H)),
            full((3 * H, H)), full((1, H)), full((H, NSEM_PAD)),
            full((1, NSEM_PAD)), full((H, F)), full((1, F)),
        ],
        out_specs=[
            full((C, 1)), eel_spec, full((C, F)), full((C, NSEM_PAD)),
        ],
        out_shape=[
            jax.ShapeDtypeStruct((C, 1), jnp.float32),
            jax.ShapeDtypeStruct((C * C, T), jnp.float32),
            jax.ShapeDtypeStruct((C, F), jnp.float32),
            jax.ShapeDtypeStruct((C, NSEM_PAD), jnp.float32),
        ],
        scratch_shapes=[
            pltpu.VMEM((C, H), jnp.float32),
            pltpu.VMEM((C, H), jnp.float32),
            pltpu.VMEM((C, H), jnp.float32),
            pltpu.VMEM((C, 1), jnp.float32),
            pltpu.VMEM((C * C, H), jnp.float32),
            pltpu.VMEM((C * C, T), jnp.float32),
            pltpu.VMEM((C, 1), jnp.float32),
            pltpu.VMEM((C, H), jnp.float32),
            pltpu.VMEM((C, H), jnp.float32),
        ],
    )(parent_feature, Wp, bp.reshape(1, H * C), We, be.reshape(1, 1),
      Wel[0:H], Wel[H:2 * H], bel.reshape(1, H), Wee, bee.reshape(1, T),
      Wne, bne, Wc, bc.reshape(1, H), wsem_p, bsem_p, Wc2, bc2.reshape(1, F))

    return (feats.reshape(1, C, F),
            sem[:, :NSEM].reshape(1, C, NSEM),
            cel.reshape(1, C, 1),
            eel.reshape(1, C, C, T))


# speedup vs baseline: 1.1094x; 1.0402x over previous
import jax
import jax.numpy as jnp
from jax.experimental import pallas as pl
from jax.experimental.pallas import tpu as pltpu

C = 128
H = 256
F = 256
T = 4
IT = 2
NSEM = 57
NSEM_PAD = 64

BI = 32
NBLK = C // BI
PF_BN = 4096
PF_ROWS = PF_BN // H
N_A = (H * C) // PF_BN
BIG = 1e30

_S_A = 0
_S_B1 = _S_A + N_A
_S_B2 = _S_B1 + 1
_S_MP0 = _S_B2 + NBLK
_S_MP1 = _S_MP0 + NBLK
_S_D = _S_MP1 + NBLK
_NSTEPS = _S_D + 1


def _dot(a, b):
    return jnp.dot(a, b, preferred_element_type=jnp.float32)


def _mp_stage(i, it, cf_s, el_s, coeff_s, cnt_s, sel_s, wne_ref, bne2_ref, out_s):
    i0 = i * BI
    cf = cf_s[...]
    a_rows = cf_s[pl.ds(i0, BI), :]
    a_msg = _dot(a_rows, wne_ref[it, 0:H, :])
    b_msg = _dot(cf, wne_ref[it, H:2 * H, :]) + bne2_ref[it:it + 1, :]
    el = el_s[pl.ds(i0 * C, BI * C), :]
    elw = _dot(el, wne_ref[it, 2 * H:3 * H, :]).reshape(BI, C, H)
    base = (elw + a_msg[:, None, :] + b_msg[None, :, :]).reshape(BI * C, H)
    coeff = coeff_s[pl.ds(i0 * C, BI * C), :]
    mfm1 = jnp.where(coeff > 0, 0.0, -1.0)
    nt = jnp.zeros((BI * C, H), dtype=jnp.float32)
    w4 = wne_ref[it, 3 * H:3 * H + T, :]
    bigrow = jnp.full((1, H), BIG, dtype=jnp.float32)
    for t in range(T):
        lhs = jnp.concatenate([coeff[:, t:t + 1], mfm1[:, t:t + 1]], axis=1)
        rhs = jnp.concatenate([w4[t:t + 1, :], bigrow], axis=0)
        z_t = _dot(lhs, rhs)
        nt = nt + jnp.maximum(base + z_t, 0.0)
    sums = _dot(sel_s[...], nt)
    denom = jnp.maximum(cnt_s[pl.ds(i0, BI), :], 1.0)
    cf_new = sums / denom
    total = jnp.sum(cnt_s[...])
    out_s[pl.ds(i0, BI), :] = jnp.where(total > 0.0, cf_new, a_rows)


def _body(parent_ref, wp_ref, bp_ref, we_ref, be_ref, wel_ref,
          bel_ref, wee_ref, bee_ref,
          wne_ref, bne2_ref,
          wc_ref, bc_ref, wsem_ref, bsem_ref, wc2_ref, bc2_ref,
          cel_ref, eel_ref, feat_ref, sem_ref,
          cf0_s, u_s, v_s, cel_s, el_s, coeff_s, cnt_s, cf1_s, cf2_s, sel_s):
    s = pl.program_id(0)

    @pl.when(s == 0)
    def _init_sel():
        rows = jax.lax.broadcasted_iota(jnp.int32, (BI, BI * C), 0)
        cols = jax.lax.broadcasted_iota(jnp.int32, (BI, BI * C), 1)
        sel_s[...] = (cols // C == rows).astype(jnp.float32)

    @pl.when(s < _S_B1)
    def _stage_a():
        o = jnp.maximum(_dot(parent_ref[...], wp_ref[...]) + bp_ref[...], 0.0)
        cf0_s[pl.ds(s * PF_ROWS, PF_ROWS), :] = o.reshape(PF_ROWS, H)

    @pl.when(s == _S_B1)
    def _stage_b1():
        cf = cf0_s[...]
        cel = _dot(cf, we_ref[...]) + be_ref[...]
        cel_s[...] = cel
        cel_ref[...] = cel
        u_s[...] = _dot(cf, wel_ref[0:H, :])
        v_s[...] = _dot(cf, wel_ref[H:2 * H, :]) + bel_ref[...]

    @pl.when((s >= _S_B2) & (s < _S_MP0))
    def _stage_b2():
        i = s - _S_B2
        i0 = i * BI
        u_b = u_s[pl.ds(i0, BI), :]
        el = jnp.maximum(u_b[:, None, :] + v_s[...][None, :, :],
                         0.0).reshape(BI * C, H)
        el_s[pl.ds(i0 * C, BI * C), :] = el
        eel = _dot(el, wee_ref[...]) + bee_ref[...]
        eel_ref[...] = eel
        cel = cel_s[...]
        celj = jnp.broadcast_to(cel.reshape(1, C, 1),
                                (BI, C, 1)).reshape(BI * C, 1)
        celi = jnp.broadcast_to(cel_s[pl.ds(i0, BI), :][:, None, :],
                                (BI, C, 1)).reshape(BI * C, 1)
        mask = (eel > 0) & (celi > 0) & (celj > 0)
        mf = mask.astype(jnp.float32)
        coeff_s[pl.ds(i0 * C, BI * C), :] = mf * eel
        s1 = jnp.sum(mf.reshape(BI, C, T), axis=2)
        cnt_s[pl.ds(i0, BI), :] = jnp.sum(s1, axis=1)[:, None]

    @pl.when((s >= _S_MP0) & (s < _S_MP1))
    def _stage_mp0():
        _mp_stage(s - _S_MP0, 0, cf0_s, el_s, coeff_s, cnt_s, sel_s,
                  wne_ref, bne2_ref, cf1_s)

    @pl.when((s >= _S_MP1) & (s < _S_D))
    def _stage_mp1():
        _mp_stage(s - _S_MP1, 1, cf1_s, el_s, coeff_s, cnt_s, sel_s,
                  wne_ref, bne2_ref, cf2_s)

    @pl.when(s == _S_D)
    def _stage_d():
        y = (_dot(cf0_s[...], wc_ref[0:H, :])
             + _dot(cf1_s[...], wc_ref[H:2 * H, :])
             + _dot(cf2_s[...], wc_ref[2 * H:3 * H, :])
             + bc_ref[...])
        y = jnp.maximum(y, 0.0)
        sem_ref[...] = _dot(y, wsem_ref[...]) + bsem_ref[...]
        feat_ref[...] = jnp.maximum(_dot(y, wc2_ref[...]) + bc2_ref[...], 0.0)


def kernel(parent_feature, Wp, bp, We, be, Wel, bel, Wee, bee, Wne, bne,
           Wc, bc, Wsem, bsem, Wc2, bc2):
    full = lambda shp: pl.BlockSpec(shp, lambda s: tuple(0 for _ in shp))
    wp_spec = pl.BlockSpec((F, PF_BN), lambda s: (0, jnp.minimum(s, N_A - 1)))
    bp_spec = pl.BlockSpec((1, PF_BN), lambda s: (0, jnp.minimum(s, N_A - 1)))
    eel_spec = pl.BlockSpec(
        (BI * C, T), lambda s: (jnp.clip(s - _S_B2, 0, NBLK - 1), 0))

    cel, eel, feats, sem = pl.pallas_call(
        _body,
        grid=(_NSTEPS,),
        in_specs=[
            full((1, F)), wp_spec, bp_spec,
            full((H, 1)), full((1, 1)), full((2 * H, H)),
            full((1, H)), full((H, T)), full((1, T)),
            full((IT, 3 * H + T, H)), full((IT, H)),
            full((3 * H, H)), full((1, H)), full((H, NSEM)),
            full((1, NSEM)), full((H, F)), full((1, F)),
        ],
        out_specs=[
            full((C, 1)), eel_spec, full((C, F)), full((C, NSEM)),
        ],
        out_shape=[
            jax.ShapeDtypeStruct((C, 1), jnp.float32),
            jax.ShapeDtypeStruct((C * C, T), jnp.float32),
            jax.ShapeDtypeStruct((C, F), jnp.float32),
            jax.ShapeDtypeStruct((C, NSEM), jnp.float32),
        ],
        scratch_shapes=[
            pltpu.VMEM((C, H), jnp.float32),
            pltpu.VMEM((C, H), jnp.float32),
            pltpu.VMEM((C, H), jnp.float32),
            pltpu.VMEM((C, 1), jnp.float32),
            pltpu.VMEM((C * C, H), jnp.float32),
            pltpu.VMEM((C * C, T), jnp.float32),
            pltpu.VMEM((C, 1), jnp.float32),
            pltpu.VMEM((C, H), jnp.float32),
            pltpu.VMEM((C, H), jnp.float32),
            pltpu.VMEM((BI, BI * C), jnp.float32),
        ],
    )(parent_feature, Wp, bp.reshape(1, H * C), We, be.reshape(1, 1),
      Wel, bel.reshape(1, H), Wee, bee.reshape(1, T),
      Wne, bne, Wc, bc.reshape(1, H), Wsem, bsem.reshape(1, NSEM), Wc2,
      bc2.reshape(1, F))

    return (feats.reshape(1, C, F),
            sem.reshape(1, C, NSEM),
            cel.reshape(1, C, 1),
            eel.reshape(1, C, C, T))
